# baseline (speedup 1.0000x reference)
import functools

import jax
import jax.numpy as jnp
from jax import lax
from jax.experimental import pallas as pl
from jax.experimental.pallas import tpu as pltpu
from jax.experimental.pallas import tpu_sc as plsc

N, D, H, G = 50000, 512, 512, 512
R = 2000

NA = 18000

TILE_ROWS = 80
CBLK = 128
NRG = 8
HC = H // 2


def _mlp_body(x_ref, wp_ref, bp_ref, wg_ref, bg_ref, out_ref):
    x = x_ref[...]
    proj = jnp.dot(x, wp_ref[...], preferred_element_type=jnp.float32) + bp_ref[...]
    gate_l = jnp.dot(x, wg_ref[...], preferred_element_type=jnp.float32) + bg_ref[...]
    out_ref[...] = jax.nn.sigmoid(gate_l) * proj


def _gated_mlp(x, wp, bp, wg, bg):
    grid = x.shape[0] // R
    return pl.pallas_call(
        _mlp_body,
        grid=(grid,),
        in_specs=[
            pl.BlockSpec((R, D), lambda i: (i, 0)),
            pl.BlockSpec((D, H), lambda i: (0, 0)),
            pl.BlockSpec((1, H), lambda i: (0, 0)),
            pl.BlockSpec((D, H), lambda i: (0, 0)),
            pl.BlockSpec((1, H), lambda i: (0, 0)),
        ],
        out_specs=pl.BlockSpec((R, H), lambda i: (i, 0)),
        out_shape=jax.ShapeDtypeStruct((x.shape[0], H), jnp.float32),
    )(x, wp, bp, wg, bg)


def _make_seg_sum(n_rows):
    n_tiles = n_rows // TILE_ROWS
    assert n_tiles * TILE_ROWS == n_rows
    cpw = (n_tiles // NRG) & ~1
    extras = n_tiles - cpw * NRG
    assert 0 <= extras <= NRG

    mesh = plsc.VectorSubcoreMesh(core_axis_name="c", subcore_axis_name="s")

    @functools.partial(
        pl.kernel,
        mesh=mesh,
        out_type=(jax.ShapeDtypeStruct((G, H), jnp.float32),
                  jax.ShapeDtypeStruct((NRG, G, H), jnp.float32)),
        scratch_types=[
            pltpu.VMEM((G, CBLK), jnp.float32),
            pltpu.VMEM((TILE_ROWS, CBLK), jnp.float32),
            pltpu.VMEM((TILE_ROWS, CBLK), jnp.float32),
            pltpu.VMEM((TILE_ROWS,), jnp.int32),
            pltpu.VMEM((TILE_ROWS,), jnp.int32),
            pltpu.VMEM((32, HC), jnp.float32),
            pltpu.VMEM((32, HC), jnp.float32),
            pltpu.SemaphoreType.DMA,
            pltpu.SemaphoreType.DMA,
            pltpu.SemaphoreType.DMA,
            pltpu.SemaphoreType.DMA,
        ],
    )
    def k(gated_hbm, ids_hbm, out_hbm, part_hbm, acc, rows0, rows1,
          ids0, ids1, av, pv, sr0, sr1, si0, si1):
        cid = lax.axis_index("c")
        sid = lax.axis_index("s")
        rg = sid // 2
        col0 = cid * HC + (sid % 2) * CBLK

        rows_b = (rows0, rows1)
        ids_b = (ids0, ids1)
        sr = (sr0, sr1)
        si = (si0, si1)

        def start(b, t):
            base = t * TILE_ROWS
            pltpu.async_copy(
                gated_hbm.at[pl.ds(base, TILE_ROWS), pl.ds(col0, CBLK)],
                rows_b[b], sr[b])
            pltpu.async_copy(ids_hbm.at[pl.ds(base, TILE_ROWS)], ids_b[b], si[b])

        def wait(b):
            pltpu.make_async_copy(
                gated_hbm.at[pl.ds(0, TILE_ROWS), pl.ds(col0, CBLK)],
                rows_b[b], sr[b]).wait()
            pltpu.make_async_copy(ids_hbm.at[pl.ds(0, TILE_ROWS)],
                                  ids_b[b], si[b]).wait()

        def compute(b):
            rv, iv = rows_b[b], ids_b[b]

            def grp(g):
                idg = iv[pl.ds(g * 16, 16)]
                first = idg[0]
                last = idg[15]

                @pl.when(first == last)
                def _uniform():
                    for k8 in range(CBLK // 16):
                        cs = pl.ds(k8 * 16, 16)
                        v = [rv[g * 16 + j, cs] for j in range(16)]
                        while len(v) > 1:
                            v = [v[i] + v[i + 1] for i in range(0, len(v), 2)]
                        plsc.addupdate(acc.at[first, cs], v[0])

                @pl.when(first != last)
                def _mixed():
                    for j in range(16):
                        sj = idg[j]
                        row = g * 16 + j
                        for k8 in range(CBLK // 16):
                            plsc.addupdate(acc.at[sj, pl.ds(k8 * 16, 16)],
                                           rv[row, pl.ds(k8 * 16, 16)])

            plsc.parallel_loop(0, TILE_ROWS // 16)(grp)

        def zr(g, carry):
            for k8 in range(CBLK // 16):
                acc[g, pl.ds(k8 * 16, 16)] = jnp.zeros((16,), jnp.float32)
            return carry

        lax.fori_loop(0, G, zr, 0)

        if cpw > 0:
            start(0, rg)
            start(1, NRG + rg)

            def outer(i2, carry):
                for b in range(2):
                    i = i2 * 2 + b
                    wait(b)
                    compute(b)

                    @pl.when(i + 2 < cpw)
                    def _():
                        start(b, (i + 2) * NRG + rg)

                return carry

            lax.fori_loop(0, cpw // 2, outer, 0)

        if extras > 0:
            @pl.when(rg < extras)
            def _extra():
                start(0, cpw * NRG + rg)
                wait(0)
                compute(0)

        pltpu.sync_copy(acc, part_hbm.at[rg, :, pl.ds(col0, CBLK)])
        plsc.subcore_barrier()

        row0 = sid * 32
        colc = cid * HC
        pltpu.sync_copy(part_hbm.at[0, pl.ds(row0, 32), pl.ds(colc, HC)], av)

        def comb(r2, carry):
            pltpu.sync_copy(part_hbm.at[r2, pl.ds(row0, 32), pl.ds(colc, HC)], pv)

            def crow(r, c2):
                for kk in range(HC // 16):
                    av[r, pl.ds(kk * 16, 16)] = (av[r, pl.ds(kk * 16, 16)]
                                                 + pv[r, pl.ds(kk * 16, 16)])
                return c2

            lax.fori_loop(0, 32, crow, 0)
            return carry

        lax.fori_loop(1, NRG, comb, 0)
        pltpu.sync_copy(av, out_hbm.at[pl.ds(row0, 32), pl.ds(colc, HC)])

    def run(gated, ids):
        out, _ = k(gated, ids)
        return out

    return run


_seg_sum_a = _make_seg_sum(NA)
_seg_sum_b = _make_seg_sum(N - NA)


def _add2_body(a_ref, b_ref, o_ref):
    o_ref[...] = a_ref[...] + b_ref[...]


def _add2(a, b):
    return pl.pallas_call(
        _add2_body,
        out_shape=jax.ShapeDtypeStruct((G, H), jnp.float32),
    )(a, b)


def kernel(node_features, node_to_graph_id, W_proj, b_proj, W_gate, b_gate):
    ids = node_to_graph_id.astype(jnp.int32)
    bp = b_proj.reshape(1, H)
    bg = b_gate.reshape(1, H)
    ga = _gated_mlp(node_features[:NA], W_proj, bp, W_gate, bg)
    pa = _seg_sum_a(ga, ids[:NA])
    gb = _gated_mlp(node_features[NA:], W_proj, bp, W_gate, bg)
    pb = _seg_sum_b(gb, ids[NA:])
    return _add2(pa, pb)


# speedup vs baseline: 1.1240x; 1.1240x over previous
import functools

import jax
import jax.numpy as jnp
from jax import lax
from jax.experimental import pallas as pl
from jax.experimental.pallas import tpu as pltpu
from jax.experimental.pallas import tpu_sc as plsc

N, D, H, G = 50000, 512, 512, 512
R = 2000

TILE_ROWS = 80
CBLK = 128
NRG = 8
HC = H // 2
NBUF = 4


def _mlp_body(x_ref, wp_ref, bp_ref, wg_ref, bg_ref, out_ref):
    x = x_ref[...]
    proj = jnp.dot(x, wp_ref[...], preferred_element_type=jnp.float32) + bp_ref[...]
    gate_l = jnp.dot(x, wg_ref[...], preferred_element_type=jnp.float32) + bg_ref[...]
    out_ref[...] = jax.nn.sigmoid(gate_l) * proj


def _gated_mlp(x, wp, bp, wg, bg):
    grid = x.shape[0] // R
    return pl.pallas_call(
        _mlp_body,
        grid=(grid,),
        in_specs=[
            pl.BlockSpec((R, D), lambda i: (i, 0)),
            pl.BlockSpec((D, H), lambda i: (0, 0)),
            pl.BlockSpec((1, H), lambda i: (0, 0)),
            pl.BlockSpec((D, H), lambda i: (0, 0)),
            pl.BlockSpec((1, H), lambda i: (0, 0)),
        ],
        out_specs=pl.BlockSpec((R, H), lambda i: (i, 0)),
        out_shape=jax.ShapeDtypeStruct((x.shape[0], H), jnp.float32),
    )(x, wp, bp, wg, bg)


def _make_seg_sum(n_rows):
    n_tiles = n_rows // TILE_ROWS
    assert n_tiles * TILE_ROWS == n_rows
    cpw = (n_tiles // NRG) & ~(NBUF - 1)
    extras = n_tiles - cpw * NRG
    extra_rounds = -(-extras // NRG)

    mesh = plsc.VectorSubcoreMesh(core_axis_name="c", subcore_axis_name="s")

    @functools.partial(
        pl.kernel,
        mesh=mesh,
        out_type=(jax.ShapeDtypeStruct((G, H), jnp.float32),
                  jax.ShapeDtypeStruct((NRG, G, H), jnp.float32)),
        scratch_types=(
            [pltpu.VMEM((G, CBLK), jnp.float32)]
            + [pltpu.VMEM((TILE_ROWS, CBLK), jnp.float32) for _ in range(NBUF)]
            + [pltpu.VMEM((TILE_ROWS,), jnp.int32) for _ in range(NBUF)]
            + [pltpu.VMEM((32, HC), jnp.float32),
               pltpu.VMEM((32, HC), jnp.float32)]
            + [pltpu.SemaphoreType.DMA for _ in range(2 * NBUF)]
        ),
    )
    def k(gated_hbm, ids_hbm, out_hbm, part_hbm, acc,
          r0, r1, r2, r3, i0, i1, i2, i3, av, pv,
          sr0, sr1, sr2, sr3, si0, si1, si2, si3):
        cid = lax.axis_index("c")
        sid = lax.axis_index("s")
        rg = sid // 2
        col0 = cid * HC + (sid % 2) * CBLK

        rows_b = (r0, r1, r2, r3)
        ids_b = (i0, i1, i2, i3)
        sr = (sr0, sr1, sr2, sr3)
        si = (si0, si1, si2, si3)

        def start(b, t):
            base = t * TILE_ROWS
            pltpu.async_copy(
                gated_hbm.at[pl.ds(base, TILE_ROWS), pl.ds(col0, CBLK)],
                rows_b[b], sr[b])
            pltpu.async_copy(ids_hbm.at[pl.ds(base, TILE_ROWS)], ids_b[b], si[b])

        def wait(b):
            pltpu.make_async_copy(
                gated_hbm.at[pl.ds(0, TILE_ROWS), pl.ds(col0, CBLK)],
                rows_b[b], sr[b]).wait()
            pltpu.make_async_copy(ids_hbm.at[pl.ds(0, TILE_ROWS)],
                                  ids_b[b], si[b]).wait()

        def compute(b):
            rv, iv = rows_b[b], ids_b[b]

            def grp(g):
                idg = iv[pl.ds(g * 16, 16)]
                first = idg[0]
                last = idg[15]

                @pl.when(first == last)
                def _uniform():
                    for k8 in range(CBLK // 16):
                        cs = pl.ds(k8 * 16, 16)
                        v = [rv[g * 16 + j, cs] for j in range(16)]
                        while len(v) > 1:
                            v = [v[i] + v[i + 1] for i in range(0, len(v), 2)]
                        plsc.addupdate(acc.at[first, cs], v[0])

                @pl.when(first != last)
                def _mixed():
                    for j in range(16):
                        sj = idg[j]
                        row = g * 16 + j
                        for k8 in range(CBLK // 16):
                            plsc.addupdate(acc.at[sj, pl.ds(k8 * 16, 16)],
                                           rv[row, pl.ds(k8 * 16, 16)])

            plsc.parallel_loop(0, TILE_ROWS // 16)(grp)

        def zr(g, carry):
            for k8 in range(CBLK // 16):
                acc[g, pl.ds(k8 * 16, 16)] = jnp.zeros((16,), jnp.float32)
            return carry

        lax.fori_loop(0, G, zr, 0)

        if cpw > 0:
            for b in range(NBUF):
                start(b, b * NRG + rg)

            def outer(iq, carry):
                for b in range(NBUF):
                    i = iq * NBUF + b
                    wait(b)
                    compute(b)

                    @pl.when(i + NBUF < cpw)
                    def _():
                        start(b, (i + NBUF) * NRG + rg)

                return carry

            lax.fori_loop(0, cpw // NBUF, outer, 0)

        for e in range(extra_rounds):
            t = (cpw + e) * NRG + rg
            if (cpw + e) * NRG + NRG <= n_tiles:
                start(0, t)
                wait(0)
                compute(0)
            else:
                @pl.when(t < n_tiles)
                def _extra():
                    start(0, t)
                    wait(0)
                    compute(0)

        pltpu.sync_copy(acc, part_hbm.at[rg, :, pl.ds(col0, CBLK)])
        plsc.subcore_barrier()

        row0 = sid * 32
        colc = cid * HC
        pltpu.sync_copy(part_hbm.at[0, pl.ds(row0, 32), pl.ds(colc, HC)], av)

        def comb(r2_, carry):
            pltpu.sync_copy(part_hbm.at[r2_, pl.ds(row0, 32), pl.ds(colc, HC)], pv)

            def crow(r, c2):
                for kk in range(HC // 16):
                    av[r, pl.ds(kk * 16, 16)] = (av[r, pl.ds(kk * 16, 16)]
                                                 + pv[r, pl.ds(kk * 16, 16)])
                return c2

            lax.fori_loop(0, 32, crow, 0)
            return carry

        lax.fori_loop(1, NRG, comb, 0)
        pltpu.sync_copy(av, out_hbm.at[pl.ds(row0, 32), pl.ds(colc, HC)])

    def run(gated, ids):
        out, _ = k(gated, ids)
        return out

    return run


_seg_sum = _make_seg_sum(N)


def kernel(node_features, node_to_graph_id, W_proj, b_proj, W_gate, b_gate):
    ids = node_to_graph_id.astype(jnp.int32)
    gated = _gated_mlp(node_features, W_proj, b_proj.reshape(1, H),
                       W_gate, b_gate.reshape(1, H))
    return _seg_sum(gated, ids)


# speedup vs baseline: 1.3046x; 1.1607x over previous
import functools

import jax
import jax.numpy as jnp
from jax import lax
from jax.experimental import pallas as pl
from jax.experimental.pallas import tpu as pltpu
from jax.experimental.pallas import tpu_sc as plsc

N, D, H, G = 50000, 512, 512, 512
R = 2000

TILE_ROWS = 80
CBLK = 128
NRG = 8
HC = H // 2
NBUF = 2


def _mlp_body(x_ref, wp_ref, bp_ref, wg_ref, bg_ref, out_ref):
    x = x_ref[...]
    proj = jnp.dot(x, wp_ref[...], preferred_element_type=jnp.float32) + bp_ref[...]
    gate_l = jnp.dot(x, wg_ref[...], preferred_element_type=jnp.float32) + bg_ref[...]
    out_ref[...] = jax.nn.sigmoid(gate_l) * proj


def _gated_mlp(x, wp, bp, wg, bg):
    grid = x.shape[0] // R
    return pl.pallas_call(
        _mlp_body,
        grid=(grid,),
        in_specs=[
            pl.BlockSpec((R, D), lambda i: (i, 0)),
            pl.BlockSpec((D, H), lambda i: (0, 0)),
            pl.BlockSpec((1, H), lambda i: (0, 0)),
            pl.BlockSpec((D, H), lambda i: (0, 0)),
            pl.BlockSpec((1, H), lambda i: (0, 0)),
        ],
        out_specs=pl.BlockSpec((R, H), lambda i: (i, 0)),
        out_shape=jax.ShapeDtypeStruct((x.shape[0], H), jnp.float32),
    )(x, wp, bp, wg, bg)


def _make_seg_sum(n_rows):
    n_tiles = n_rows // TILE_ROWS
    assert n_tiles * TILE_ROWS == n_rows
    cpw = (n_tiles // NRG) & ~(NBUF - 1)
    extras = n_tiles - cpw * NRG
    extra_rounds = -(-extras // NRG)

    mesh = plsc.VectorSubcoreMesh(core_axis_name="c", subcore_axis_name="s")

    @functools.partial(
        pl.kernel,
        mesh=mesh,
        out_type=(jax.ShapeDtypeStruct((G, H), jnp.float32),
                  jax.ShapeDtypeStruct((NRG, G, H), jnp.float32)),
        scratch_types=(
            [pltpu.VMEM((G, CBLK), jnp.float32)]
            + [pltpu.VMEM((TILE_ROWS, CBLK), jnp.float32) for _ in range(NBUF)]
            + [pltpu.VMEM((TILE_ROWS,), jnp.int32) for _ in range(NBUF)]
            + [pltpu.VMEM((32, HC), jnp.float32),
               pltpu.VMEM((32, HC), jnp.float32)]
            + [pltpu.SemaphoreType.DMA for _ in range(2 * NBUF)]
        ),
    )
    def k(gated_hbm, ids_hbm, out_hbm, part_hbm, acc,
          r0, r1, i0, i1, av, pv,
          sr0, sr1, si0, si1):
        cid = lax.axis_index("c")
        sid = lax.axis_index("s")
        rg = sid // 2
        col0 = cid * HC + (sid % 2) * CBLK

        rows_b = (r0, r1)
        ids_b = (i0, i1)
        sr = (sr0, sr1)
        si = (si0, si1)

        def start(b, t):
            base = t * TILE_ROWS
            pltpu.async_copy(
                gated_hbm.at[pl.ds(base, TILE_ROWS), pl.ds(col0, CBLK)],
                rows_b[b], sr[b])
            pltpu.async_copy(ids_hbm.at[pl.ds(base, TILE_ROWS)], ids_b[b], si[b])

        def wait(b):
            pltpu.make_async_copy(
                gated_hbm.at[pl.ds(0, TILE_ROWS), pl.ds(col0, CBLK)],
                rows_b[b], sr[b]).wait()
            pltpu.make_async_copy(ids_hbm.at[pl.ds(0, TILE_ROWS)],
                                  ids_b[b], si[b]).wait()

        def compute(b):
            rv, iv = rows_b[b], ids_b[b]

            def grp(g):
                idg = iv[pl.ds(g * 16, 16)]
                first = idg[0]
                last = idg[15]

                @pl.when(first == last)
                def _uniform():
                    for k8 in range(CBLK // 16):
                        cs = pl.ds(k8 * 16, 16)
                        v = [rv[g * 16 + j, cs] for j in range(16)]
                        while len(v) > 1:
                            v = [v[i] + v[i + 1] for i in range(0, len(v), 2)]
                        plsc.addupdate(acc.at[first, cs], v[0])

                @pl.when(first != last)
                def _mixed():
                    for j in range(16):
                        sj = idg[j]
                        row = g * 16 + j
                        for k8 in range(CBLK // 16):
                            plsc.addupdate(acc.at[sj, pl.ds(k8 * 16, 16)],
                                           rv[row, pl.ds(k8 * 16, 16)])

            plsc.parallel_loop(0, TILE_ROWS // 16)(grp)

        def zr(g, carry):
            for k8 in range(CBLK // 16):
                acc[g, pl.ds(k8 * 16, 16)] = jnp.zeros((16,), jnp.float32)
            return carry

        lax.fori_loop(0, G, zr, 0)

        if cpw > 0:
            for b in range(NBUF):
                start(b, b * NRG + rg)

            def outer(iq, carry):
                for b in range(NBUF):
                    i = iq * NBUF + b
                    wait(b)
                    compute(b)

                    @pl.when(i + NBUF < cpw)
                    def _():
                        start(b, (i + NBUF) * NRG + rg)

                return carry

            lax.fori_loop(0, cpw // NBUF, outer, 0)

        for e in range(extra_rounds):
            t = (cpw + e) * NRG + rg
            if (cpw + e) * NRG + NRG <= n_tiles:
                start(0, t)
                wait(0)
                compute(0)
            else:
                @pl.when(t < n_tiles)
                def _extra():
                    start(0, t)
                    wait(0)
                    compute(0)

        pltpu.sync_copy(acc, part_hbm.at[rg, :, pl.ds(col0, CBLK)])
        plsc.subcore_barrier()

        row0 = sid * 32
        colc = cid * HC
        pltpu.sync_copy(part_hbm.at[0, pl.ds(row0, 32), pl.ds(colc, HC)], av)

        def comb(r2_, carry):
            pltpu.sync_copy(part_hbm.at[r2_, pl.ds(row0, 32), pl.ds(colc, HC)], pv)

            def crow(r, c2):
                for kk in range(HC // 16):
                    av[r, pl.ds(kk * 16, 16)] = (av[r, pl.ds(kk * 16, 16)]
                                                 + pv[r, pl.ds(kk * 16, 16)])
                return c2

            lax.fori_loop(0, 32, crow, 0)
            return carry

        lax.fori_loop(1, NRG, comb, 0)
        pltpu.sync_copy(av, out_hbm.at[pl.ds(row0, 32), pl.ds(colc, HC)])

    def run(gated, ids):
        out, _ = k(gated, ids)
        return out

    return run


_seg_sum = _make_seg_sum(N)


def kernel(node_features, node_to_graph_id, W_proj, b_proj, W_gate, b_gate):
    ids = node_to_graph_id.astype(jnp.int32)
    gated = _gated_mlp(node_features, W_proj, b_proj.reshape(1, H),
                       W_gate, b_gate.reshape(1, H))
    return _seg_sum(gated, ids)


# speedup vs baseline: 1.3132x; 1.0066x over previous
import functools

import jax
import jax.numpy as jnp
from jax import lax
from jax.experimental import pallas as pl
from jax.experimental.pallas import tpu as pltpu
from jax.experimental.pallas import tpu_sc as plsc

N, D, H, G = 50000, 512, 512, 512
R = 2000

TILE_ROWS = 80
CBLK = 128
NRG = 8
HC = H // 2
NBUF = 2


def _mlp_body(x_ref, wp_ref, bp_ref, wg_ref, bg_ref, out_ref):
    x = x_ref[...]
    proj = jnp.dot(x, wp_ref[...], preferred_element_type=jnp.float32) + bp_ref[...]
    gate_l = jnp.dot(x, wg_ref[...], preferred_element_type=jnp.float32) + bg_ref[...]
    out_ref[...] = jax.nn.sigmoid(gate_l) * proj


def _gated_mlp(x, wp, bp, wg, bg):
    grid = x.shape[0] // R
    return pl.pallas_call(
        _mlp_body,
        grid=(grid,),
        in_specs=[
            pl.BlockSpec((R, D), lambda i: (i, 0)),
            pl.BlockSpec((D, H), lambda i: (0, 0)),
            pl.BlockSpec((1, H), lambda i: (0, 0)),
            pl.BlockSpec((D, H), lambda i: (0, 0)),
            pl.BlockSpec((1, H), lambda i: (0, 0)),
        ],
        out_specs=pl.BlockSpec((R, H), lambda i: (i, 0)),
        out_shape=jax.ShapeDtypeStruct((x.shape[0], H), jnp.float32),
    )(x, wp, bp, wg, bg)


def _make_seg_sum(n_rows):
    n_tiles = n_rows // TILE_ROWS
    assert n_tiles * TILE_ROWS == n_rows
    cpw = (n_tiles // NRG) & ~(NBUF - 1)
    extras = n_tiles - cpw * NRG
    extra_rounds = -(-extras // NRG)

    mesh = plsc.VectorSubcoreMesh(core_axis_name="c", subcore_axis_name="s")

    @functools.partial(
        pl.kernel,
        mesh=mesh,
        out_type=(jax.ShapeDtypeStruct((G, H), jnp.float32),
                  jax.ShapeDtypeStruct((NRG, G, H), jnp.float32)),
        scratch_types=(
            [pltpu.VMEM((G, CBLK), jnp.float32)]
            + [pltpu.VMEM((TILE_ROWS, CBLK), jnp.float32) for _ in range(NBUF)]
            + [pltpu.VMEM((TILE_ROWS,), jnp.int32) for _ in range(NBUF)]
            + [pltpu.VMEM((32, HC), jnp.float32),
               pltpu.VMEM((32, HC), jnp.float32)]
            + [pltpu.SemaphoreType.DMA for _ in range(2 * NBUF)]
        ),
    )
    def k(gated_hbm, ids_hbm, out_hbm, part_hbm, acc,
          r0, r1, i0, i1, av, pv,
          sr0, sr1, si0, si1):
        cid = lax.axis_index("c")
        sid = lax.axis_index("s")
        rg = sid // 2
        col0 = cid * HC + (sid % 2) * CBLK

        rows_b = (r0, r1)
        ids_b = (i0, i1)
        sr = (sr0, sr1)
        si = (si0, si1)

        def start(b, t):
            base = t * TILE_ROWS
            pltpu.async_copy(
                gated_hbm.at[pl.ds(base, TILE_ROWS), pl.ds(col0, CBLK)],
                rows_b[b], sr[b])
            pltpu.async_copy(ids_hbm.at[pl.ds(base, TILE_ROWS)], ids_b[b], si[b])

        def wait(b):
            pltpu.make_async_copy(
                gated_hbm.at[pl.ds(0, TILE_ROWS), pl.ds(col0, CBLK)],
                rows_b[b], sr[b]).wait()
            pltpu.make_async_copy(ids_hbm.at[pl.ds(0, TILE_ROWS)],
                                  ids_b[b], si[b]).wait()

        def compute(b):
            rv, iv = rows_b[b], ids_b[b]

            def grp(g):
                idg = iv[pl.ds(g * 16, 16)]
                first = idg[0]
                last = idg[15]

                @pl.when(first == last)
                def _uniform():
                    for k8 in range(CBLK // 16):
                        cs = pl.ds(k8 * 16, 16)
                        v = [rv[g * 16 + j, cs] for j in range(16)]
                        while len(v) > 1:
                            v = [v[i] + v[i + 1] for i in range(0, len(v), 2)]
                        plsc.addupdate(acc.at[first, cs], v[0])

                @pl.when(first != last)
                def _mixed():
                    for j in range(16):
                        sj = idg[j]
                        row = g * 16 + j
                        for k8 in range(CBLK // 16):
                            plsc.addupdate(acc.at[sj, pl.ds(k8 * 16, 16)],
                                           rv[row, pl.ds(k8 * 16, 16)])

            plsc.parallel_loop(0, TILE_ROWS // 16)(grp)

        if cpw > 0:
            for b in range(NBUF):
                start(b, b * NRG + rg)

        def zr(g, carry):
            for k8 in range(CBLK // 16):
                acc[g, pl.ds(k8 * 16, 16)] = jnp.zeros((16,), jnp.float32)
            return carry

        lax.fori_loop(0, G, zr, 0)

        if cpw > 0:
            def outer(iq, carry):
                for b in range(NBUF):
                    i = iq * NBUF + b
                    wait(b)
                    compute(b)

                    @pl.when(i + NBUF < cpw)
                    def _():
                        start(b, (i + NBUF) * NRG + rg)

                return carry

            lax.fori_loop(0, cpw // NBUF, outer, 0)

        for e in range(extra_rounds):
            t = (cpw + e) * NRG + rg
            if (cpw + e) * NRG + NRG <= n_tiles:
                start(0, t)
                wait(0)
                compute(0)
            else:
                @pl.when(t < n_tiles)
                def _extra():
                    start(0, t)
                    wait(0)
                    compute(0)

        pltpu.sync_copy(acc, part_hbm.at[rg, :, pl.ds(col0, CBLK)])
        plsc.subcore_barrier()

        row0 = sid * 32
        colc = cid * HC
        pltpu.sync_copy(part_hbm.at[0, pl.ds(row0, 32), pl.ds(colc, HC)], av)

        def comb(r2_, carry):
            pltpu.sync_copy(part_hbm.at[r2_, pl.ds(row0, 32), pl.ds(colc, HC)], pv)

            def crow(r, c2):
                for kk in range(HC // 16):
                    av[r, pl.ds(kk * 16, 16)] = (av[r, pl.ds(kk * 16, 16)]
                                                 + pv[r, pl.ds(kk * 16, 16)])
                return c2

            lax.fori_loop(0, 32, crow, 0)
            return carry

        lax.fori_loop(1, NRG, comb, 0)
        pltpu.sync_copy(av, out_hbm.at[pl.ds(row0, 32), pl.ds(colc, HC)])

    def run(gated, ids):
        out, _ = k(gated, ids)
        return out

    return run


_seg_sum = _make_seg_sum(N)


def kernel(node_features, node_to_graph_id, W_proj, b_proj, W_gate, b_gate):
    ids = node_to_graph_id.astype(jnp.int32)
    gated = _gated_mlp(node_features, W_proj, b_proj.reshape(1, H),
                       W_gate, b_gate.reshape(1, H))
    return _seg_sum(gated, ids)


# speedup vs baseline: 1.9661x; 1.4972x over previous
import functools

import jax
import jax.numpy as jnp
from jax import lax
from jax.experimental import pallas as pl
from jax.experimental.pallas import tpu as pltpu
from jax.experimental.pallas import tpu_sc as plsc

N, D, H, G = 50000, 512, 512, 512
R = 2000

TILE_ROWS = 80
CBLK = 128
NRG = 8
HC = H // 2
NBUF = 2


def _mlp_body(x_ref, wp_ref, bp_ref, wg_ref, bg_ref, out_ref):
    x = x_ref[...]
    proj = jnp.dot(x, wp_ref[...], preferred_element_type=jnp.float32) + bp_ref[...]
    gate_l = jnp.dot(x, wg_ref[...], preferred_element_type=jnp.float32) + bg_ref[...]
    out_ref[...] = jax.nn.sigmoid(gate_l) * proj


def _gated_mlp(x, wp, bp, wg, bg, off_blocks=0, n_rows=None):
    n_rows = x.shape[0] if n_rows is None else n_rows
    grid = n_rows // R
    return pl.pallas_call(
        _mlp_body,
        grid=(grid,),
        in_specs=[
            pl.BlockSpec((R, D), lambda i: (i + off_blocks, 0)),
            pl.BlockSpec((D, H), lambda i: (0, 0)),
            pl.BlockSpec((1, H), lambda i: (0, 0)),
            pl.BlockSpec((D, H), lambda i: (0, 0)),
            pl.BlockSpec((1, H), lambda i: (0, 0)),
        ],
        out_specs=pl.BlockSpec((R, H), lambda i: (i, 0)),
        out_shape=jax.ShapeDtypeStruct((n_rows, H), jnp.float32),
    )(x, wp, bp, wg, bg)


def _make_seg_sum(n_rows):
    n_tiles = n_rows // TILE_ROWS
    assert n_tiles * TILE_ROWS == n_rows
    cpw = (n_tiles // NRG) & ~(NBUF - 1)
    extras = n_tiles - cpw * NRG
    extra_rounds = -(-extras // NRG)

    mesh = plsc.VectorSubcoreMesh(core_axis_name="c", subcore_axis_name="s")

    @functools.partial(
        pl.kernel,
        mesh=mesh,
        out_type=(jax.ShapeDtypeStruct((G, H), jnp.float32),
                  jax.ShapeDtypeStruct((NRG, G, H), jnp.float32)),
        scratch_types=(
            [pltpu.VMEM((G, CBLK), jnp.float32)]
            + [pltpu.VMEM((TILE_ROWS, CBLK), jnp.float32) for _ in range(NBUF)]
            + [pltpu.VMEM((TILE_ROWS,), jnp.int32) for _ in range(NBUF)]
            + [pltpu.VMEM((32, HC), jnp.float32),
               pltpu.VMEM((32, HC), jnp.float32)]
            + [pltpu.SemaphoreType.DMA for _ in range(2 * NBUF)]
        ),
    )
    def k(gated_hbm, ids_hbm, out_hbm, part_hbm, acc,
          r0, r1, i0, i1, av, pv,
          sr0, sr1, si0, si1):
        cid = lax.axis_index("c")
        sid = lax.axis_index("s")
        rg = sid // 2
        col0 = cid * HC + (sid % 2) * CBLK

        rows_b = (r0, r1)
        ids_b = (i0, i1)
        sr = (sr0, sr1)
        si = (si0, si1)

        def start(b, t):
            base = t * TILE_ROWS
            pltpu.async_copy(
                gated_hbm.at[pl.ds(base, TILE_ROWS), pl.ds(col0, CBLK)],
                rows_b[b], sr[b])
            pltpu.async_copy(ids_hbm.at[pl.ds(base, TILE_ROWS)], ids_b[b], si[b])

        def wait(b):
            pltpu.make_async_copy(
                gated_hbm.at[pl.ds(0, TILE_ROWS), pl.ds(col0, CBLK)],
                rows_b[b], sr[b]).wait()
            pltpu.make_async_copy(ids_hbm.at[pl.ds(0, TILE_ROWS)],
                                  ids_b[b], si[b]).wait()

        def compute(b):
            rv, iv = rows_b[b], ids_b[b]

            def grp(g):
                idg = iv[pl.ds(g * 16, 16)]
                first = idg[0]
                last = idg[15]

                @pl.when(first == last)
                def _uniform():
                    for k8 in range(CBLK // 16):
                        cs = pl.ds(k8 * 16, 16)
                        v = [rv[g * 16 + j, cs] for j in range(16)]
                        while len(v) > 1:
                            v = [v[i] + v[i + 1] for i in range(0, len(v), 2)]
                        plsc.addupdate(acc.at[first, cs], v[0])

                @pl.when(first != last)
                def _mixed():
                    for j in range(16):
                        sj = idg[j]
                        row = g * 16 + j
                        for k8 in range(CBLK // 16):
                            plsc.addupdate(acc.at[sj, pl.ds(k8 * 16, 16)],
                                           rv[row, pl.ds(k8 * 16, 16)])

            plsc.parallel_loop(0, TILE_ROWS // 16)(grp)

        if cpw > 0:
            for b in range(NBUF):
                start(b, b * NRG + rg)

        def zr(g, carry):
            for k8 in range(CBLK // 16):
                acc[g, pl.ds(k8 * 16, 16)] = jnp.zeros((16,), jnp.float32)
            return carry

        lax.fori_loop(0, G, zr, 0)

        if cpw > 0:
            def outer(iq, carry):
                for b in range(NBUF):
                    i = iq * NBUF + b
                    wait(b)
                    compute(b)

                    @pl.when(i + NBUF < cpw)
                    def _():
                        start(b, (i + NBUF) * NRG + rg)

                return carry

            lax.fori_loop(0, cpw // NBUF, outer, 0)

        for e in range(extra_rounds):
            t = (cpw + e) * NRG + rg
            if (cpw + e) * NRG + NRG <= n_tiles:
                start(0, t)
                wait(0)
                compute(0)
            else:
                @pl.when(t < n_tiles)
                def _extra():
                    start(0, t)
                    wait(0)
                    compute(0)

        pltpu.sync_copy(acc, part_hbm.at[rg, :, pl.ds(col0, CBLK)])
        plsc.subcore_barrier()

        row0 = sid * 32
        colc = cid * HC
        pltpu.sync_copy(part_hbm.at[0, pl.ds(row0, 32), pl.ds(colc, HC)], av)

        def comb(r2_, carry):
            pltpu.sync_copy(part_hbm.at[r2_, pl.ds(row0, 32), pl.ds(colc, HC)], pv)

            def crow(r, c2):
                for kk in range(HC // 16):
                    av[r, pl.ds(kk * 16, 16)] = (av[r, pl.ds(kk * 16, 16)]
                                                 + pv[r, pl.ds(kk * 16, 16)])
                return c2

            lax.fori_loop(0, 32, crow, 0)
            return carry

        lax.fori_loop(1, NRG, comb, 0)
        pltpu.sync_copy(av, out_hbm.at[pl.ds(row0, 32), pl.ds(colc, HC)])

    def run(gated, ids):
        out, _ = k(gated, ids)
        return out

    return run


NB = 26000
NA = N - NB
_seg_sum = _make_seg_sum(NB)


def _fused_body(ids_ref, x_ref, wp_ref, bp_ref, wg_ref, bg_ref, out_ref):
    i = pl.program_id(0)
    x = x_ref[...]
    proj = jnp.dot(x, wp_ref[...], preferred_element_type=jnp.float32) + bp_ref[...]
    gate_l = jnp.dot(x, wg_ref[...], preferred_element_type=jnp.float32) + bg_ref[...]
    gated = jax.nn.sigmoid(gate_l) * proj
    onehot_t = (jax.lax.broadcasted_iota(jnp.int32, (G, R), 0) == ids_ref[0]
                ).astype(jnp.float32)
    partial = jnp.dot(onehot_t, gated, preferred_element_type=jnp.float32)

    @pl.when(i == 0)
    def _init():
        out_ref[...] = jnp.zeros_like(out_ref)

    out_ref[...] += partial


def _fused_onehot(x, ids3, wp, bp, wg, bg, off_blocks, n_rows):
    grid = n_rows // R
    return pl.pallas_call(
        _fused_body,
        grid=(grid,),
        in_specs=[
            pl.BlockSpec((1, 1, R), lambda i: (i, 0, 0)),
            pl.BlockSpec((R, D), lambda i: (i + off_blocks, 0)),
            pl.BlockSpec((D, H), lambda i: (0, 0)),
            pl.BlockSpec((1, H), lambda i: (0, 0)),
            pl.BlockSpec((D, H), lambda i: (0, 0)),
            pl.BlockSpec((1, H), lambda i: (0, 0)),
        ],
        out_specs=pl.BlockSpec((G, H), lambda i: (0, 0)),
        out_shape=jax.ShapeDtypeStruct((G, H), jnp.float32),
    )(ids3, x, wp, bp, wg, bg)


def _add2_body(a_ref, b_ref, o_ref):
    o_ref[...] = a_ref[...] + b_ref[...]


def _add2(a, b):
    return pl.pallas_call(
        _add2_body,
        out_shape=jax.ShapeDtypeStruct((G, H), jnp.float32),
    )(a, b)


def kernel(node_features, node_to_graph_id, W_proj, b_proj, W_gate, b_gate):
    ids = node_to_graph_id.astype(jnp.int32)
    bp = b_proj.reshape(1, H)
    bg = b_gate.reshape(1, H)
    gb = _gated_mlp(node_features, W_proj, bp, W_gate, bg,
                    off_blocks=0, n_rows=NB)
    p_sc = _seg_sum(gb, ids[:NB])
    ids3 = ids[NB:].reshape(NA // R, 1, R)
    p_tc = _fused_onehot(node_features, ids3, W_proj, bp, W_gate, bg,
                         off_blocks=NB // R, n_rows=NA)
    return _add2(p_sc, p_tc)


# speedup vs baseline: 2.2264x; 1.1324x over previous
import functools

import jax
import jax.numpy as jnp
from jax import lax
from jax.experimental import pallas as pl
from jax.experimental.pallas import tpu as pltpu
from jax.experimental.pallas import tpu_sc as plsc

N, D, H, G = 50000, 512, 512, 512
R = 2000

TILE_ROWS = 80
CBLK = 128
NRG = 8
HC = H // 2
NBUF = 2


def _mlp_body(x_ref, wp_ref, bp_ref, wg_ref, bg_ref, out_ref):
    x = x_ref[...]
    proj = jnp.dot(x, wp_ref[...], preferred_element_type=jnp.float32) + bp_ref[...]
    gate_l = jnp.dot(x, wg_ref[...], preferred_element_type=jnp.float32) + bg_ref[...]
    out_ref[...] = jax.nn.sigmoid(gate_l) * proj


def _gated_mlp(x, wp, bp, wg, bg, off_blocks=0, n_rows=None):
    n_rows = x.shape[0] if n_rows is None else n_rows
    grid = n_rows // R
    return pl.pallas_call(
        _mlp_body,
        grid=(grid,),
        in_specs=[
            pl.BlockSpec((R, D), lambda i: (i + off_blocks, 0)),
            pl.BlockSpec((D, H), lambda i: (0, 0)),
            pl.BlockSpec((1, H), lambda i: (0, 0)),
            pl.BlockSpec((D, H), lambda i: (0, 0)),
            pl.BlockSpec((1, H), lambda i: (0, 0)),
        ],
        out_specs=pl.BlockSpec((R, H), lambda i: (i, 0)),
        out_shape=jax.ShapeDtypeStruct((n_rows, H), jnp.float32),
    )(x, wp, bp, wg, bg)


def _make_seg_sum(n_rows):
    n_tiles = n_rows // TILE_ROWS
    assert n_tiles * TILE_ROWS == n_rows
    cpw = (n_tiles // NRG) & ~(NBUF - 1)
    extras = n_tiles - cpw * NRG
    extra_rounds = -(-extras // NRG)

    mesh = plsc.VectorSubcoreMesh(core_axis_name="c", subcore_axis_name="s")

    @functools.partial(
        pl.kernel,
        mesh=mesh,
        out_type=(jax.ShapeDtypeStruct((G, H), jnp.float32),
                  jax.ShapeDtypeStruct((NRG, G, H), jnp.float32)),
        scratch_types=(
            [pltpu.VMEM((G, CBLK), jnp.float32)]
            + [pltpu.VMEM((TILE_ROWS, CBLK), jnp.float32) for _ in range(NBUF)]
            + [pltpu.VMEM((TILE_ROWS,), jnp.int32) for _ in range(NBUF)]
            + [pltpu.VMEM((32, HC), jnp.float32),
               pltpu.VMEM((32, HC), jnp.float32)]
            + [pltpu.SemaphoreType.DMA for _ in range(2 * NBUF)]
        ),
    )
    def k(gated_hbm, ids_hbm, out_hbm, part_hbm, acc,
          r0, r1, i0, i1, av, pv,
          sr0, sr1, si0, si1):
        cid = lax.axis_index("c")
        sid = lax.axis_index("s")
        rg = sid // 2
        col0 = cid * HC + (sid % 2) * CBLK

        rows_b = (r0, r1)
        ids_b = (i0, i1)
        sr = (sr0, sr1)
        si = (si0, si1)

        def start(b, t):
            base = t * TILE_ROWS
            pltpu.async_copy(
                gated_hbm.at[pl.ds(base, TILE_ROWS), pl.ds(col0, CBLK)],
                rows_b[b], sr[b])
            pltpu.async_copy(ids_hbm.at[pl.ds(base, TILE_ROWS)], ids_b[b], si[b])

        def wait(b):
            pltpu.make_async_copy(
                gated_hbm.at[pl.ds(0, TILE_ROWS), pl.ds(col0, CBLK)],
                rows_b[b], sr[b]).wait()
            pltpu.make_async_copy(ids_hbm.at[pl.ds(0, TILE_ROWS)],
                                  ids_b[b], si[b]).wait()

        def compute(b):
            rv, iv = rows_b[b], ids_b[b]

            def grp(g):
                idg = iv[pl.ds(g * 16, 16)]
                first = idg[0]
                last = idg[15]

                @pl.when(first == last)
                def _uniform():
                    for k8 in range(CBLK // 16):
                        cs = pl.ds(k8 * 16, 16)
                        v = [rv[g * 16 + j, cs] for j in range(16)]
                        while len(v) > 1:
                            v = [v[i] + v[i + 1] for i in range(0, len(v), 2)]
                        plsc.addupdate(acc.at[first, cs], v[0])

                @pl.when(first != last)
                def _mixed():
                    for j in range(16):
                        sj = idg[j]
                        row = g * 16 + j
                        for k8 in range(CBLK // 16):
                            plsc.addupdate(acc.at[sj, pl.ds(k8 * 16, 16)],
                                           rv[row, pl.ds(k8 * 16, 16)])

            plsc.parallel_loop(0, TILE_ROWS // 16)(grp)

        if cpw > 0:
            for b in range(NBUF):
                start(b, b * NRG + rg)

        def zr(g, carry):
            for k8 in range(CBLK // 16):
                acc[g, pl.ds(k8 * 16, 16)] = jnp.zeros((16,), jnp.float32)
            return carry

        lax.fori_loop(0, G, zr, 0)

        if cpw > 0:
            def outer(iq, carry):
                for b in range(NBUF):
                    i = iq * NBUF + b
                    wait(b)
                    compute(b)

                    @pl.when(i + NBUF < cpw)
                    def _():
                        start(b, (i + NBUF) * NRG + rg)

                return carry

            lax.fori_loop(0, cpw // NBUF, outer, 0)

        for e in range(extra_rounds):
            t = (cpw + e) * NRG + rg
            if (cpw + e) * NRG + NRG <= n_tiles:
                start(0, t)
                wait(0)
                compute(0)
            else:
                @pl.when(t < n_tiles)
                def _extra():
                    start(0, t)
                    wait(0)
                    compute(0)

        pltpu.sync_copy(acc, part_hbm.at[rg, :, pl.ds(col0, CBLK)])
        plsc.subcore_barrier()

        row0 = sid * 32
        colc = cid * HC
        pltpu.sync_copy(part_hbm.at[0, pl.ds(row0, 32), pl.ds(colc, HC)], av)

        def comb(r2_, carry):
            pltpu.sync_copy(part_hbm.at[r2_, pl.ds(row0, 32), pl.ds(colc, HC)], pv)

            def crow(r, c2):
                for kk in range(HC // 16):
                    av[r, pl.ds(kk * 16, 16)] = (av[r, pl.ds(kk * 16, 16)]
                                                 + pv[r, pl.ds(kk * 16, 16)])
                return c2

            lax.fori_loop(0, 32, crow, 0)
            return carry

        lax.fori_loop(1, NRG, comb, 0)
        pltpu.sync_copy(av, out_hbm.at[pl.ds(row0, 32), pl.ds(colc, HC)])

    def run(gated, ids):
        out, _ = k(gated, ids)
        return out

    return run


NB = 20000
NA = N - NB
_seg_sum = _make_seg_sum(NB)


def _fused_body(ids_ref, x_ref, wp_ref, bp_ref, wg_ref, bg_ref, out_ref):
    i = pl.program_id(0)
    x = x_ref[...]
    proj = jnp.dot(x, wp_ref[...], preferred_element_type=jnp.float32) + bp_ref[...]
    gate_l = jnp.dot(x, wg_ref[...], preferred_element_type=jnp.float32) + bg_ref[...]
    gated = jax.nn.sigmoid(gate_l) * proj
    onehot_t = (jax.lax.broadcasted_iota(jnp.int32, (G, R), 0) == ids_ref[0]
                ).astype(jnp.float32)
    partial = jnp.dot(onehot_t, gated, preferred_element_type=jnp.float32)

    @pl.when(i == 0)
    def _init():
        out_ref[...] = jnp.zeros_like(out_ref)

    out_ref[...] += partial


def _fused_onehot(x, ids3, wp, bp, wg, bg, off_blocks, n_rows):
    grid = n_rows // R
    return pl.pallas_call(
        _fused_body,
        grid=(grid,),
        in_specs=[
            pl.BlockSpec((1, 1, R), lambda i: (i, 0, 0)),
            pl.BlockSpec((R, D), lambda i: (i + off_blocks, 0)),
            pl.BlockSpec((D, H), lambda i: (0, 0)),
            pl.BlockSpec((1, H), lambda i: (0, 0)),
            pl.BlockSpec((D, H), lambda i: (0, 0)),
            pl.BlockSpec((1, H), lambda i: (0, 0)),
        ],
        out_specs=pl.BlockSpec((G, H), lambda i: (0, 0)),
        out_shape=jax.ShapeDtypeStruct((G, H), jnp.float32),
    )(ids3, x, wp, bp, wg, bg)


def _add2_body(a_ref, b_ref, o_ref):
    o_ref[...] = a_ref[...] + b_ref[...]


def _add2(a, b):
    return pl.pallas_call(
        _add2_body,
        out_shape=jax.ShapeDtypeStruct((G, H), jnp.float32),
    )(a, b)


def kernel(node_features, node_to_graph_id, W_proj, b_proj, W_gate, b_gate):
    ids = node_to_graph_id.astype(jnp.int32)
    bp = b_proj.reshape(1, H)
    bg = b_gate.reshape(1, H)
    gb = _gated_mlp(node_features, W_proj, bp, W_gate, bg,
                    off_blocks=0, n_rows=NB)
    p_sc = _seg_sum(gb, ids[:NB])
    ids3 = ids[NB:].reshape(NA // R, 1, R)
    p_tc = _fused_onehot(node_features, ids3, W_proj, bp, W_gate, bg,
                         off_blocks=NB // R, n_rows=NA)
    return _add2(p_sc, p_tc)
